# baseline (speedup 1.0000x reference)
import functools
import jax
import jax.numpy as jnp
from jax import lax
from jax.experimental import pallas as pl
from jax.experimental.pallas import tpu as pltpu
from jax.experimental.pallas import tpu_sc as plsc

_NBUF = 4


def kernel(x, pos_emb):
    seq_len = x.shape[1]
    d_model = pos_emb.shape[1]
    info = plsc.get_sparse_core_info()
    nc = info.num_cores
    rows_per_c = seq_len // nc
    chunk = 256
    nchunks = rows_per_c // chunk
    mesh = plsc.VectorSubcoreMesh(core_axis_name="c", subcore_axis_name="s")

    @functools.partial(
        pl.kernel,
        mesh=mesh,
        out_type=jax.ShapeDtypeStruct((seq_len, d_model), jnp.float32),
        scratch_types=[
            pltpu.VMEM_SHARED((_NBUF, chunk, d_model), jnp.float32),
            pltpu.SemaphoreType.DMA((_NBUF,)),
            pltpu.SemaphoreType.DMA((_NBUF,)),
        ],
    )
    def sc_copy(table_hbm, out_hbm, buf, in_sems, out_sems):
        cid = lax.axis_index("c")
        sid = lax.axis_index("s")
        base = cid * rows_per_c

        def in_copy(j):
            return pltpu.make_async_copy(
                table_hbm.at[pl.ds(base + j * chunk, chunk), :],
                buf.at[j % _NBUF],
                in_sems.at[j % _NBUF],
            )

        def out_copy(j):
            return pltpu.make_async_copy(
                buf.at[j % _NBUF],
                out_hbm.at[pl.ds(base + j * chunk, chunk), :],
                out_sems.at[j % _NBUF],
            )

        @pl.when(sid == 0)
        def _():
            for j in range(_NBUF):
                in_copy(j).start()
            for j in range(nchunks):
                in_copy(j).wait()
                out_copy(j).start()
                if j + _NBUF < nchunks:
                    out_copy(j).wait()
                    in_copy(j + _NBUF).start()
            for j in range(nchunks - _NBUF, nchunks):
                out_copy(j).wait()

    return sc_copy(pos_emb)
